# baseline (speedup 1.0000x reference)
import functools

import jax
import jax.numpy as jnp
from jax.experimental import pallas as pl
from jax.experimental.pallas import tpu as pltpu

_NBUF = 3
_NCORES = 2


def _moe_body(eids_ref, ew_ref, x_ref, w1_any, w3_any, w2_any, out_ref,
              w1b, w3b, w2b, cnt, base, uexp, starts, scnt, srows, sems,
              *, top_k, n_experts):
    P = eids_ref.shape[0]
    core = pl.program_id(0)
    out_ref[...] = jnp.zeros_like(out_ref)
    zero = jnp.int32(0)

    def clear_body(e, c):
        cnt[e] = zero
        return c
    jax.lax.fori_loop(0, n_experts, clear_body, zero)

    def count_body(i, c):
        cnt[eids_ref[i]] += 1
        return c
    jax.lax.fori_loop(0, P, count_body, zero)

    def seg_scan(e, carry):
        d, pos = carry
        c = cnt[e]

        @pl.when(c > 0)
        def _():
            uexp[d] = e
            starts[d] = pos
            scnt[d] = c
        base[e] = pos
        return jnp.where(c > 0, d + 1, d), pos + c

    d, _ = jax.lax.fori_loop(0, n_experts, seg_scan, (zero, zero))

    def scatter_body(i, c):
        e = eids_ref[i]
        b = base[e]
        srows[b] = i
        base[e] = b + 1
        return c
    jax.lax.fori_loop(0, P, scatter_body, zero)

    nseg = (d - core + _NCORES - 1) // _NCORES

    def _copies(m, slot):
        e = uexp[core + m * _NCORES]
        return (
            pltpu.make_async_copy(w1_any.at[e], w1b.at[slot], sems.at[slot, 0]),
            pltpu.make_async_copy(w3_any.at[e], w3b.at[slot], sems.at[slot, 1]),
            pltpu.make_async_copy(w2_any.at[e], w2b.at[slot], sems.at[slot, 2]),
        )

    for ss in range(_NBUF):
        @pl.when(ss < nseg)
        def _start():
            for c in _copies(ss, ss):
                c.start()

    def seg_body(m, carry):
        slot = jax.lax.rem(m, _NBUF)
        for c in _copies(m, slot):
            c.wait()

        j = core + m * _NCORES
        start = starts[j]

        def row_body(r, carry2):
            p = srows[r]
            t = p // top_k
            xrow = x_ref[0, pl.ds(t, 1), :]
            g = jnp.dot(xrow, w1b[slot], preferred_element_type=jnp.float32)
            u = jnp.dot(xrow, w3b[slot], preferred_element_type=jnp.float32)
            h = (g * jax.nn.sigmoid(g)) * u
            o = jnp.dot(h, w2b[slot], preferred_element_type=jnp.float32)
            out_ref[0, pl.ds(t, 1), :] += ew_ref[p] * o
            return carry2

        jax.lax.fori_loop(start, start + scnt[j], row_body, zero)

        @pl.when(m + _NBUF < nseg)
        def _next():
            for c in _copies(m + _NBUF, slot):
                c.start()
        return carry

    jax.lax.fori_loop(0, nseg, seg_body, zero)


def kernel(x, expert_indices, expert_weights, w1_stacked, w2_stacked, w3_stacked):
    B, H = x.shape
    K = expert_indices.shape[1]
    E, _, I = w1_stacked.shape
    P = B * K

    eids = expert_indices.reshape(P).astype(jnp.int32)
    ew = expert_weights.reshape(P)
    x3 = x.reshape(1, B, H)

    grid_spec = pltpu.PrefetchScalarGridSpec(
        num_scalar_prefetch=2,
        grid=(_NCORES,),
        in_specs=[
            pl.BlockSpec((1, B, H), lambda i, *_: (0, 0, 0)),
            pl.BlockSpec(memory_space=pl.ANY),
            pl.BlockSpec(memory_space=pl.ANY),
            pl.BlockSpec(memory_space=pl.ANY),
        ],
        out_specs=pl.BlockSpec((1, B, H), lambda i, *_: (i, 0, 0)),
        scratch_shapes=[
            pltpu.VMEM((_NBUF, H, I), jnp.float32),
            pltpu.VMEM((_NBUF, H, I), jnp.float32),
            pltpu.VMEM((_NBUF, I, H), jnp.float32),
            pltpu.SMEM((E,), jnp.int32),
            pltpu.SMEM((E,), jnp.int32),
            pltpu.SMEM((P,), jnp.int32),
            pltpu.SMEM((P,), jnp.int32),
            pltpu.SMEM((P,), jnp.int32),
            pltpu.SMEM((P,), jnp.int32),
            pltpu.SemaphoreType.DMA((_NBUF, 3)),
        ],
    )
    fn = pl.pallas_call(
        functools.partial(_moe_body, top_k=K, n_experts=E),
        grid_spec=grid_spec,
        out_shape=jax.ShapeDtypeStruct((_NCORES, B, H), jnp.float32),
        compiler_params=pltpu.CompilerParams(
            dimension_semantics=("parallel",)),
    )
    parts = fn(eids, ew, x3, w1_stacked, w3_stacked, w2_stacked)
    return parts.sum(axis=0)


# speedup vs baseline: 1.0471x; 1.0471x over previous
import functools

import jax
import jax.numpy as jnp
from jax.experimental import pallas as pl
from jax.experimental.pallas import tpu as pltpu

_NBUF = 3


def _moe_body(eids_ref, ew_ref, x_ref, w1_any, w3_any, w2_any, out_ref,
              w1b, w3b, w2b, cnt, base, uexp, starts, scnt, srows, sems,
              *, top_k, n_experts):
    P = eids_ref.shape[0]
    out_ref[...] = jnp.zeros_like(out_ref)
    zero = jnp.int32(0)

    def clear_body(e, c):
        cnt[e] = zero
        return c
    jax.lax.fori_loop(0, n_experts, clear_body, zero)

    def count_body(i, c):
        cnt[eids_ref[i]] += 1
        return c
    jax.lax.fori_loop(0, P, count_body, zero)

    def seg_scan(e, carry):
        d, pos = carry
        c = cnt[e]

        @pl.when(c > 0)
        def _():
            uexp[d] = e
            starts[d] = pos
            scnt[d] = c
        base[e] = pos
        return jnp.where(c > 0, d + 1, d), pos + c

    d, _ = jax.lax.fori_loop(0, n_experts, seg_scan, (zero, zero))

    def scatter_body(i, c):
        e = eids_ref[i]
        b = base[e]
        srows[b] = i
        base[e] = b + 1
        return c
    jax.lax.fori_loop(0, P, scatter_body, zero)

    def _copies(j, slot):
        e = uexp[j]
        h2 = w1_any.shape[1] // 2
        i2 = w2_any.shape[1] // 2
        return (
            pltpu.make_async_copy(w1_any.at[e, pl.ds(0, h2)],
                                  w1b.at[slot, pl.ds(0, h2)], sems.at[slot, 0]),
            pltpu.make_async_copy(w1_any.at[e, pl.ds(h2, h2)],
                                  w1b.at[slot, pl.ds(h2, h2)], sems.at[slot, 1]),
            pltpu.make_async_copy(w3_any.at[e, pl.ds(0, h2)],
                                  w3b.at[slot, pl.ds(0, h2)], sems.at[slot, 2]),
            pltpu.make_async_copy(w3_any.at[e, pl.ds(h2, h2)],
                                  w3b.at[slot, pl.ds(h2, h2)], sems.at[slot, 3]),
            pltpu.make_async_copy(w2_any.at[e, pl.ds(0, i2)],
                                  w2b.at[slot, pl.ds(0, i2)], sems.at[slot, 4]),
            pltpu.make_async_copy(w2_any.at[e, pl.ds(i2, i2)],
                                  w2b.at[slot, pl.ds(i2, i2)], sems.at[slot, 5]),
        )

    for jj in range(_NBUF):
        @pl.when(jj < d)
        def _start():
            for c in _copies(jj, jj):
                c.start()

    def seg_body(j, carry):
        slot = jax.lax.rem(j, _NBUF)
        for c in _copies(j, slot):
            c.wait()

        start = starts[j]

        def row_body(r, carry2):
            p = srows[r]
            t = p // top_k
            xrow = x_ref[pl.ds(t, 1), :]
            g = jnp.dot(xrow, w1b[slot], preferred_element_type=jnp.float32)
            u = jnp.dot(xrow, w3b[slot], preferred_element_type=jnp.float32)
            h = (g * jax.nn.sigmoid(g)) * u
            o = jnp.dot(h, w2b[slot], preferred_element_type=jnp.float32)
            out_ref[pl.ds(t, 1), :] += ew_ref[p] * o
            return carry2

        jax.lax.fori_loop(start, start + scnt[j], row_body, zero)

        @pl.when(j + _NBUF < d)
        def _next():
            for c in _copies(j + _NBUF, slot):
                c.start()
        return carry

    jax.lax.fori_loop(0, d, seg_body, zero)


def kernel(x, expert_indices, expert_weights, w1_stacked, w2_stacked, w3_stacked):
    B, H = x.shape
    K = expert_indices.shape[1]
    E, _, I = w1_stacked.shape
    P = B * K

    eids = expert_indices.reshape(P).astype(jnp.int32)
    ew = expert_weights.reshape(P)

    grid_spec = pltpu.PrefetchScalarGridSpec(
        num_scalar_prefetch=2,
        grid=(1,),
        in_specs=[
            pl.BlockSpec((B, H), lambda i, *_: (0, 0)),
            pl.BlockSpec(memory_space=pl.ANY),
            pl.BlockSpec(memory_space=pl.ANY),
            pl.BlockSpec(memory_space=pl.ANY),
        ],
        out_specs=pl.BlockSpec((B, H), lambda i, *_: (0, 0)),
        scratch_shapes=[
            pltpu.VMEM((_NBUF, H, I), jnp.float32),
            pltpu.VMEM((_NBUF, H, I), jnp.float32),
            pltpu.VMEM((_NBUF, I, H), jnp.float32),
            pltpu.SMEM((E,), jnp.int32),
            pltpu.SMEM((E,), jnp.int32),
            pltpu.SMEM((P,), jnp.int32),
            pltpu.SMEM((P,), jnp.int32),
            pltpu.SMEM((P,), jnp.int32),
            pltpu.SMEM((P,), jnp.int32),
            pltpu.SemaphoreType.DMA((_NBUF, 6)),
        ],
    )
    fn = pl.pallas_call(
        functools.partial(_moe_body, top_k=K, n_experts=E),
        grid_spec=grid_spec,
        out_shape=jax.ShapeDtypeStruct((B, H), jnp.float32),
    )
    return fn(eids, ew, x, w1_stacked, w3_stacked, w2_stacked)
